# baseline (speedup 1.0000x reference)
import jax
import jax.numpy as jnp
from jax import lax
from jax.experimental import pallas as pl
from jax.experimental.pallas import tpu as pltpu
from jax.experimental.pallas import tpu_sc as plsc

B = 128
V = 100000
VB = 2048
NB = (V + VB - 1) // VB
L = 16


def _fused_body(log_ref, pen_ref, idx_ref, out_ref, maxv, argv):
    j = pl.program_id(0)
    pen = pen_ref[...]
    out_ref[...] = pen
    scaled = log_ref[...] * pen
    col = lax.broadcasted_iota(jnp.int32, (B, VB), 1) + j * VB
    scaled = jnp.where(col < V, scaled, -jnp.inf)
    bmax = jnp.max(scaled, axis=1, keepdims=True)
    barg = (jnp.argmax(scaled, axis=1).astype(jnp.int32) + j * VB).reshape(B, 1)

    @pl.when(j == 0)
    def _():
        maxv[...] = bmax
        argv[...] = barg

    @pl.when(j > 0)
    def _():
        upd = bmax > maxv[...]
        maxv[...] = jnp.where(upd, bmax, maxv[...])
        argv[...] = jnp.where(upd, barg, argv[...])

    @pl.when(j == NB - 1)
    def _():
        idx_ref[...] = argv[...]


def _fused_pass(logits, repeat_penality):
    return pl.pallas_call(
        _fused_body,
        grid=(NB,),
        in_specs=[
            pl.BlockSpec((B, VB), lambda j: (0, j)),
            pl.BlockSpec((B, VB), lambda j: (0, j)),
        ],
        out_specs=[
            pl.BlockSpec((B, 1), lambda j: (0, 0)),
            pl.BlockSpec((B, VB), lambda j: (0, j)),
        ],
        out_shape=[
            jax.ShapeDtypeStruct((B, 1), jnp.int32),
            jax.ShapeDtypeStruct((B, V), jnp.float32),
        ],
        scratch_shapes=[
            pltpu.VMEM((B, 1), jnp.float32),
            pltpu.VMEM((B, 1), jnp.int32),
        ],
        compiler_params=pltpu.CompilerParams(
            dimension_semantics=("arbitrary",),
        ),
    )(logits, repeat_penality)


def _sc_scatter_body(pen_ref, idx_hbm, pv_hbm, idx_v, flat_v, vals_v, pv_v, sem):
    cid = lax.axis_index("c")
    sid = lax.axis_index("s")

    @pl.when(jnp.logical_and(cid == 0, sid == 0))
    def _():
        pltpu.sync_copy(idx_hbm, idx_v)
        pltpu.sync_copy(pv_hbm, pv_v)
        for k in range(B // L):
            rows = lax.iota(jnp.int32, L) + (k * L)
            flat_v[0, pl.ds(k * L, L)] = rows * V + idx_v[pl.ds(k * L, L)]
        pltpu.async_copy(pen_ref.at[flat_v.at[0]], vals_v, sem).wait()
        for k in range(B // L):
            vals_v[pl.ds(k * L, L)] = vals_v[pl.ds(k * L, L)] * pv_v[...]
        pltpu.async_copy(vals_v, pen_ref.at[flat_v.at[0]], sem).wait()


import functools


@functools.cache
def _make_sc_scatter():
    mesh = plsc.VectorSubcoreMesh(core_axis_name="c", subcore_axis_name="s")
    return pl.kernel(
        _sc_scatter_body,
        out_type=(),
        mesh=mesh,
        scratch_types=[
            pltpu.VMEM((B,), jnp.int32),
            pltpu.VMEM((1, B), jnp.int32),
            pltpu.VMEM((B,), jnp.float32),
            pltpu.VMEM((L,), jnp.float32),
            pltpu.SemaphoreType.DMA,
        ],
    )


def kernel(logits, repeat_penality, penality_value):
    idx, pen_out = _fused_pass(logits, repeat_penality)
    pen_flat_ref = jax.new_ref(pen_out.reshape(B * V))
    pv16 = jnp.full((L,), penality_value, dtype=jnp.float32)
    _make_sc_scatter()(pen_flat_ref, idx.reshape(B), pv16)
    return idx, pen_flat_ref[...].reshape(B, V)


# speedup vs baseline: 1.6063x; 1.6063x over previous
import jax
import jax.numpy as jnp
from jax import lax
from jax.experimental import pallas as pl
from jax.experimental.pallas import tpu as pltpu

B = 128
V = 100000
VB = 2048
NB = (V + VB - 1) // VB


def _body(pv_ref, log_ref, pen_ref, idx_ref, out_ref, maxv, argv, cache):
    j = pl.program_id(0)

    @pl.when(j < NB)
    def _phase_a():
        pen = pen_ref[...]
        cache[:, pl.ds(j * VB, VB)] = pen
        scaled = log_ref[...] * pen
        col = lax.broadcasted_iota(jnp.int32, (B, VB), 1) + j * VB
        scaled = jnp.where(col < V, scaled, -jnp.inf)
        bmax = jnp.max(scaled, axis=1, keepdims=True)
        barg = (jnp.argmax(scaled, axis=1).astype(jnp.int32) + j * VB).reshape(
            B, 1
        )

        @pl.when(j == 0)
        def _():
            maxv[...] = bmax
            argv[...] = barg

        @pl.when(j > 0)
        def _():
            upd = bmax > maxv[...]
            maxv[...] = jnp.where(upd, bmax, maxv[...])
            argv[...] = jnp.where(upd, barg, argv[...])

        @pl.when(j == NB - 1)
        def _():
            idx_ref[...] = argv[...]

    @pl.when(j >= NB)
    def _phase_b():
        jb = j - NB
        pen = cache[:, pl.ds(jb * VB, VB)]
        col = lax.broadcasted_iota(jnp.int32, (B, VB), 1) + jb * VB
        hit = col == argv[...]
        out_ref[...] = jnp.where(hit, pen * pv_ref[0, 0], pen)


def kernel(logits, repeat_penality, penality_value):
    idx, pen_out = pl.pallas_call(
        _body,
        grid=(2 * NB,),
        in_specs=[
            pl.BlockSpec(memory_space=pltpu.SMEM),
            pl.BlockSpec((B, VB), lambda j: (0, jnp.minimum(j, NB - 1))),
            pl.BlockSpec((B, VB), lambda j: (0, jnp.minimum(j, NB - 1))),
        ],
        out_specs=[
            pl.BlockSpec((B, 1), lambda j: (0, 0)),
            pl.BlockSpec((B, VB), lambda j: (0, jnp.maximum(j - NB, 0))),
        ],
        out_shape=[
            jax.ShapeDtypeStruct((B, 1), jnp.int32),
            jax.ShapeDtypeStruct((B, V), jnp.float32),
        ],
        scratch_shapes=[
            pltpu.VMEM((B, 1), jnp.float32),
            pltpu.VMEM((B, 1), jnp.int32),
            pltpu.VMEM((B, NB * VB), jnp.float32),
        ],
        compiler_params=pltpu.CompilerParams(
            dimension_semantics=("arbitrary",),
        ),
    )(penality_value.reshape(1, 1), logits, repeat_penality)
    return idx, pen_out
